# Initial kernel scaffold; baseline (speedup 1.0000x reference)
import jax
import jax.numpy as jnp
from jax.experimental import pallas as pl


def kernel(obs, rnn_states, masks, params, row, col, eps):
    raise NotImplementedError("write your pallas kernel here")


# speedup vs baseline: 22.2680x; 22.2680x over previous
import functools

import jax
import jax.numpy as jnp
from jax.experimental import pallas as pl
from jax.experimental.pallas import tpu as pltpu

_A = 64
_T = 128
_EQU = 3
_INV = 16
_H = 64
_NL = 2
_NN = _A * _T
_LOG_SQRT_2PI = 0.9189385332046727


def _silu(v):
    return v * jax.nn.sigmoid(v)


def _egnn_block_kernel(x0_ref, hin_ref, eps_ref, embwt_ref, embb_ref,
                       a1t_ref, b1t_ref, c1_ref, be1_ref, w2t_ref, be2_ref,
                       wc1t_ref, bc1_ref, wc2_ref, wn1at_ref, wn1bt_ref,
                       bn1_ref, wn2t_ref, bn2_ref, logstd_ref,
                       act_ref, lp_ref):
    f32 = jnp.float32
    x = x0_ref[...]
    h = jnp.dot(hin_ref[...], embwt_ref[...],
                preferred_element_type=f32) + embb_ref[...]

    ii = jax.lax.broadcasted_iota(jnp.int32, (_A, _A), 0)
    jj = jax.lax.broadcasted_iota(jnp.int32, (_A, _A), 1)
    mask = (ii != jj).astype(f32)[:, :, None]

    for l in range(_NL):
        cd = x[:, None, :] - x[None, :, :]
        radial = jnp.sum(cd * cd, axis=-1, keepdims=True)
        norm = jnp.sqrt(radial) + 1e-8
        cdn = cd / norm

        hA = jnp.dot(h, a1t_ref[l], preferred_element_type=f32) + be1_ref[l]
        hB = jnp.dot(h, b1t_ref[l], preferred_element_type=f32)
        pre = hA[:, None, :] + hB[None, :, :] + radial * c1_ref[l]
        m1 = _silu(pre).reshape(_A * _A, _H)
        m2 = _silu(jnp.dot(m1, w2t_ref[l],
                           preferred_element_type=f32) + be2_ref[l])
        cm = _silu(jnp.dot(m2, wc1t_ref[l],
                           preferred_element_type=f32) + bc1_ref[l])
        csc = jnp.tanh(jnp.sum(cm * wc2_ref[l], axis=-1, keepdims=True))
        c3 = csc.reshape(_A, _A, 1) * mask
        s = jnp.sum(cdn * c3, axis=1)
        x = x + s * (1.0 / 63.0)

        am = jnp.sum(m2.reshape(_A, _A, _H) * mask, axis=1)
        o = _silu(jnp.dot(h, wn1at_ref[l], preferred_element_type=f32)
                  + jnp.dot(am, wn1bt_ref[l], preferred_element_type=f32)
                  + bn1_ref[l])
        h = h + jnp.dot(o, wn2t_ref[l], preferred_element_type=f32) + bn2_ref[l]

    e = eps_ref[...]
    logstd = logstd_ref[...]
    act_ref[...] = x + jnp.exp(logstd) * e
    lp_ref[...] = -0.5 * (e * e) - logstd - _LOG_SQRT_2PI


@jax.jit
def _run(x0, hin, eps, weights):
    def blk(shape):
        nd = len(shape)
        return pl.BlockSpec(shape, lambda i, _n=nd: (0,) * _n)

    in_specs = [
        pl.BlockSpec((_A, _EQU), lambda i: (i, 0)),
        pl.BlockSpec((_A, _INV), lambda i: (i, 0)),
        pl.BlockSpec((_A, _EQU), lambda i: (i, 0)),
    ] + [blk(w.shape) for w in weights]
    out_specs = [pl.BlockSpec((_A, _EQU), lambda i: (i, 0))] * 2
    out_shape = [jax.ShapeDtypeStruct((_NN, _EQU), jnp.float32)] * 2

    act, lp = pl.pallas_call(
        _egnn_block_kernel,
        grid=(_T,),
        in_specs=in_specs,
        out_specs=out_specs,
        out_shape=out_shape,
        compiler_params=pltpu.CompilerParams(
            dimension_semantics=("arbitrary",)),
    )(x0, hin, eps, *weights)
    return act, lp


def kernel(obs, rnn_states, masks, params, row, col, eps):
    L = params['layers']
    H = _H

    def stk(f):
        return jnp.stack([f(p) for p in L])

    weights = [
        params['emb_W'].T,
        params['emb_b'].reshape(1, H),
        stk(lambda p: p['We1'][:, :H].T),
        stk(lambda p: p['We1'][:, H:2 * H].T),
        stk(lambda p: p['We1'][:, 2 * H].reshape(1, H)),
        stk(lambda p: p['be1'].reshape(1, H)),
        stk(lambda p: p['We2'].T),
        stk(lambda p: p['be2'].reshape(1, H)),
        stk(lambda p: p['Wc1'].T),
        stk(lambda p: p['bc1'].reshape(1, H)),
        stk(lambda p: p['Wc2'].reshape(1, H)),
        stk(lambda p: p['Wn1'][:, :H].T),
        stk(lambda p: p['Wn1'][:, H:].T),
        stk(lambda p: p['bn1'].reshape(1, H)),
        stk(lambda p: p['Wn2'].T),
        stk(lambda p: p['bn2'].reshape(1, H)),
        params['log_std'].reshape(1, _EQU),
    ]
    x0 = obs[:, :_EQU]
    hin = obs[:, _EQU:]
    act, lp = _run(x0, hin, eps, weights)
    return (act.reshape(_T, _A, _EQU),
            lp.reshape(_T, _A, _EQU),
            rnn_states)


# speedup vs baseline: 25.3830x; 1.1399x over previous
import functools

import jax
import jax.numpy as jnp
from jax.experimental import pallas as pl
from jax.experimental.pallas import tpu as pltpu

_A = 64
_T = 128
_EQU = 3
_INV = 16
_H = 64
_NL = 2
_NN = _A * _T
_P = _A * _A
_LOG_SQRT_2PI = 0.9189385332046727


def _silu(v):
    return v * (0.5 * jnp.tanh(0.5 * v) + 0.5)


def _egnn_block_kernel(x0_ref, hin_ref, eps_ref, sisj_ref, smask_ref,
                       embwt_ref, embb_ref,
                       a1t_ref, b1t_ref, c1_ref, be1_ref, w2t_ref, be2_ref,
                       wc1t_ref, bc1_ref, wc2_ref, wn1at_ref, wn1bt_ref,
                       bn1_ref, wn2t_ref, bn2_ref, logstd_ref,
                       act_ref, lp_ref):
    f32 = jnp.float32
    x = x0_ref[...]
    h = jnp.dot(hin_ref[...], embwt_ref[...],
                preferred_element_type=f32) + embb_ref[...]
    sisj = sisj_ref[...]
    smask = smask_ref[...]

    for l in range(_NL):
        cd = x[:, None, :] - x[None, :, :]
        radial = jnp.sum(cd * cd, axis=-1, keepdims=True)
        norm = jnp.sqrt(radial) + 1e-8
        cdn = cd / norm

        hA = jnp.dot(h, a1t_ref[l], preferred_element_type=f32) + be1_ref[l]
        hB = jnp.dot(h, b1t_ref[l], preferred_element_type=f32)
        hab = jnp.concatenate([hA, hB], axis=0)
        pre = (jnp.dot(sisj, hab, preferred_element_type=f32)
               + radial.reshape(_P, 1) * c1_ref[l])
        m1 = _silu(pre)
        m2 = _silu(jnp.dot(m1, w2t_ref[l],
                           preferred_element_type=f32) + be2_ref[l])
        cm = _silu(jnp.dot(m2, wc1t_ref[l],
                           preferred_element_type=f32) + bc1_ref[l])
        csc = jnp.tanh(jnp.sum(cm * wc2_ref[l], axis=-1, keepdims=True))
        s = jnp.sum(cdn * csc.reshape(_A, _A, 1), axis=1)
        x = x + s * (1.0 / 63.0)

        am = jnp.dot(smask, m2, preferred_element_type=f32)
        o = _silu(jnp.dot(h, wn1at_ref[l], preferred_element_type=f32)
                  + jnp.dot(am, wn1bt_ref[l], preferred_element_type=f32)
                  + bn1_ref[l])
        h = h + jnp.dot(o, wn2t_ref[l], preferred_element_type=f32) + bn2_ref[l]

    e = eps_ref[...]
    logstd = logstd_ref[...]
    act_ref[...] = x + jnp.exp(logstd) * e
    lp_ref[...] = -0.5 * (e * e) - logstd - _LOG_SQRT_2PI


@jax.jit
def _run(x0, hin, eps, consts):
    def blk(shape):
        nd = len(shape)
        return pl.BlockSpec(shape, lambda i, _n=nd: (0,) * _n)

    in_specs = [
        pl.BlockSpec((_A, _EQU), lambda i: (i, 0)),
        pl.BlockSpec((_A, _INV), lambda i: (i, 0)),
        pl.BlockSpec((_A, _EQU), lambda i: (i, 0)),
    ] + [blk(w.shape) for w in consts]
    out_specs = [pl.BlockSpec((_A, _EQU), lambda i: (i, 0))] * 2
    out_shape = [jax.ShapeDtypeStruct((_NN, _EQU), jnp.float32)] * 2

    act, lp = pl.pallas_call(
        _egnn_block_kernel,
        grid=(_T,),
        in_specs=in_specs,
        out_specs=out_specs,
        out_shape=out_shape,
        compiler_params=pltpu.CompilerParams(
            dimension_semantics=("arbitrary",)),
    )(x0, hin, eps, *consts)
    return act, lp


def kernel(obs, rnn_states, masks, params, row, col, eps):
    L = params['layers']
    H = _H

    def stk(f):
        return jnp.stack([f(p) for p in L])

    pr = jnp.arange(_P, dtype=jnp.int32)
    ia = jnp.arange(_A, dtype=jnp.int32)
    si = (pr[:, None] // _A == ia[None, :]).astype(jnp.float32)
    sj = (pr[:, None] % _A == ia[None, :]).astype(jnp.float32)
    sisj = jnp.concatenate([si, sj], axis=1)
    smask = (si * (1.0 - sj)).T

    consts = [
        sisj, smask,
        params['emb_W'].T,
        params['emb_b'].reshape(1, H),
        stk(lambda p: p['We1'][:, :H].T),
        stk(lambda p: p['We1'][:, H:2 * H].T),
        stk(lambda p: p['We1'][:, 2 * H].reshape(1, H)),
        stk(lambda p: p['be1'].reshape(1, H)),
        stk(lambda p: p['We2'].T),
        stk(lambda p: p['be2'].reshape(1, H)),
        stk(lambda p: p['Wc1'].T),
        stk(lambda p: p['bc1'].reshape(1, H)),
        stk(lambda p: p['Wc2'].reshape(1, H)),
        stk(lambda p: p['Wn1'][:, :H].T),
        stk(lambda p: p['Wn1'][:, H:].T),
        stk(lambda p: p['bn1'].reshape(1, H)),
        stk(lambda p: p['Wn2'].T),
        stk(lambda p: p['bn2'].reshape(1, H)),
        params['log_std'].reshape(1, _EQU),
    ]
    x0 = obs[:, :_EQU]
    hin = obs[:, _EQU:]
    act, lp = _run(x0, hin, eps, consts)
    return (act.reshape(_T, _A, _EQU),
            lp.reshape(_T, _A, _EQU),
            rnn_states)


# speedup vs baseline: 37.8768x; 1.4922x over previous
import functools

import jax
import jax.numpy as jnp
from jax.experimental import pallas as pl
from jax.experimental.pallas import tpu as pltpu

_A = 64
_T = 128
_EQU = 3
_INV = 16
_H = 64
_NL = 2
_NN = _A * _T
_P = _A * _A
_LOG_SQRT_2PI = 0.9189385332046727


def _hsilu(t):
    return t * jnp.tanh(t) + t


def _egnn_block_kernel(x0_ref, hin_ref, eps_ref, sisj_ref, smask_ref,
                       embwt_ref, embb_ref,
                       a1t_ref, b1t_ref, c1_ref, w2t_ref, be2_ref,
                       wc1t_ref, bc1_ref, wc2_ref, wn1at_ref, wn1bt_ref,
                       bn1_ref, wn2t_ref, bn2_ref, logstd_ref,
                       act_ref, lp_ref):
    f32 = jnp.float32
    x = x0_ref[0]
    h = jnp.dot(hin_ref[...], embwt_ref[...],
                preferred_element_type=f32) + embb_ref[...]
    sisj = sisj_ref[...]
    smask = smask_ref[...]

    for l in range(_NL):
        cd = x[:, :, None] - x[:, None, :]
        radial = jnp.sum(cd * cd, axis=0)
        norm = jnp.sqrt(radial) + 1e-8
        cdn = cd / norm

        hA = jnp.dot(h, a1t_ref[l], preferred_element_type=f32) + c1_ref[l, 1:]
        hB = jnp.dot(h, b1t_ref[l], preferred_element_type=f32)
        hab = jnp.concatenate([hA, hB], axis=0)
        pre = (jnp.dot(sisj, hab, preferred_element_type=f32)
                   .reshape(_A, _A, _H)
               + radial[:, :, None] * c1_ref[l, 0])
        m1 = _hsilu(pre).reshape(_P, _H)
        m2 = _hsilu(jnp.dot(m1, w2t_ref[l],
                            preferred_element_type=f32) + be2_ref[l])
        cm = _hsilu(jnp.dot(m2, wc1t_ref[l],
                            preferred_element_type=f32) + bc1_ref[l])
        csc = jnp.tanh(jnp.sum(cm.reshape(_A, _A, _H) * wc2_ref[l],
                               axis=-1))
        s = jnp.sum(cdn * csc, axis=-1)
        x = x + s * (1.0 / 63.0)

        am = jnp.dot(smask, m2, preferred_element_type=f32)
        o = _hsilu(jnp.dot(h, wn1at_ref[l], preferred_element_type=f32)
                   + jnp.dot(am, wn1bt_ref[l], preferred_element_type=f32)
                   + bn1_ref[l])
        h = h + jnp.dot(o, wn2t_ref[l], preferred_element_type=f32) + bn2_ref[l]

    e = eps_ref[0]
    logstd = logstd_ref[...]
    act_ref[0] = x + jnp.exp(logstd) * e
    lp_ref[0] = -0.5 * (e * e) - logstd - _LOG_SQRT_2PI


@jax.jit
def _run(x0, hin, eps, consts):
    def blk(shape):
        nd = len(shape)
        return pl.BlockSpec(shape, lambda i, _n=nd: (0,) * _n)

    in_specs = [
        pl.BlockSpec((1, _EQU, _A), lambda i: (i, 0, 0)),
        pl.BlockSpec((_A, _INV), lambda i: (i, 0)),
        pl.BlockSpec((1, _EQU, _A), lambda i: (i, 0, 0)),
    ] + [blk(w.shape) for w in consts]
    out_specs = [pl.BlockSpec((1, _EQU, _A), lambda i: (i, 0, 0))] * 2
    out_shape = [jax.ShapeDtypeStruct((_T, _EQU, _A), jnp.float32)] * 2

    act, lp = pl.pallas_call(
        _egnn_block_kernel,
        grid=(_T,),
        in_specs=in_specs,
        out_specs=out_specs,
        out_shape=out_shape,
        compiler_params=pltpu.CompilerParams(
            dimension_semantics=("arbitrary",)),
    )(x0, hin, eps, *consts)
    return act, lp


def kernel(obs, rnn_states, masks, params, row, col, eps):
    L = params['layers']
    H = _H

    def stk(f):
        return jnp.stack([f(p) for p in L])

    pr = jnp.arange(_P, dtype=jnp.int32)
    ia = jnp.arange(_A, dtype=jnp.int32)
    si = (pr[:, None] // _A == ia[None, :]).astype(jnp.float32)
    sj = (pr[:, None] % _A == ia[None, :]).astype(jnp.float32)
    sisj = jnp.concatenate([si, sj], axis=1)
    smask = (si * (1.0 - sj)).T

    consts = [
        sisj, smask,
        params['emb_W'].T,
        params['emb_b'].reshape(1, H),
        stk(lambda p: 0.5 * p['We1'][:, :H].T),
        stk(lambda p: 0.5 * p['We1'][:, H:2 * H].T),
        stk(lambda p: jnp.stack([0.5 * p['We1'][:, 2 * H],
                                 0.5 * p['be1']])),
        stk(lambda p: 0.5 * p['We2'].T),
        stk(lambda p: 0.5 * p['be2'].reshape(1, H)),
        stk(lambda p: 0.5 * p['Wc1'].T),
        stk(lambda p: 0.5 * p['bc1'].reshape(1, H)),
        stk(lambda p: p['Wc2'].reshape(1, H)),
        stk(lambda p: 0.5 * p['Wn1'][:, :H].T),
        stk(lambda p: 0.5 * p['Wn1'][:, H:].T),
        stk(lambda p: 0.5 * p['bn1'].reshape(1, H)),
        stk(lambda p: p['Wn2'].T),
        stk(lambda p: p['bn2'].reshape(1, H)),
        params['log_std'].reshape(_EQU, 1),
    ]
    x0 = obs[:, :_EQU].reshape(_T, _A, _EQU).transpose(0, 2, 1)
    epsT = eps.reshape(_T, _A, _EQU).transpose(0, 2, 1)
    hin = obs[:, _EQU:]
    act, lp = _run(x0, hin, epsT, consts)
    return (act.transpose(0, 2, 1),
            lp.transpose(0, 2, 1),
            rnn_states)
